# Initial kernel scaffold; baseline (speedup 1.0000x reference)
import jax
import jax.numpy as jnp
from jax.experimental import pallas as pl


def kernel(x, edge_index, edge_weight, edge_attr, W1, W2, b2, Wm1, bm1, Wm2, bm2, Wl, bl):
    raise NotImplementedError("write your pallas kernel here")


# speedup vs baseline: 3.5381x; 3.5381x over previous
import functools

import jax
import jax.numpy as jnp
import numpy as np
from jax import lax
from jax.experimental import pallas as pl
from jax.experimental.pallas import tpu as pltpu
from jax.experimental.pallas import tpu_sc as plsc

N_NODES = 10000
N_EDGES = 320000
HIDDEN = 128
N_GAUSS = 16
SHIFT = float(np.log(2.0))

NC = 2
NS = 16
NW = NC * NS
EPW = N_EDGES // NW
CHUNK = 80
NCHUNK = EPW // CHUNK
NRC = N_NODES // CHUNK


def _ssp(v):
    return jnp.maximum(v, 0.0) + jnp.log1p(jnp.exp(-jnp.abs(v))) - SHIFT


def _h_body(x_ref, w1_ref, o_ref):
    o_ref[...] = lax.dot_general(x_ref[...], w1_ref[...],
                                 (((1,), (1,)), ((), ())),
                                 preferred_element_type=jnp.float32)


def _compute_h(x, W1):
    bn = 2000
    return pl.pallas_call(
        _h_body,
        grid=(N_NODES // bn,),
        in_specs=[pl.BlockSpec((bn, HIDDEN), lambda i: (i, 0)),
                  pl.BlockSpec((HIDDEN, HIDDEN), lambda i: (0, 0))],
        out_specs=pl.BlockSpec((bn, HIDDEN), lambda i: (i, 0)),
        out_shape=jax.ShapeDtypeStruct((N_NODES, HIDDEN), jnp.float32),
    )(x, W1)


def _mlp_body(ea_ref, wm1_ref, bm1_ref, wm2_ref, bm2_ref, o_ref):
    a = lax.dot_general(ea_ref[...], wm1_ref[...], (((1,), (1,)), ((), ())),
                        preferred_element_type=jnp.float32)
    a = _ssp(a + bm1_ref[...])
    o = lax.dot_general(a, wm2_ref[...], (((1,), (1,)), ((), ())),
                        preferred_element_type=jnp.float32)
    o_ref[...] = o + bm2_ref[...]


def _compute_mlp(edge_attr, Wm1, bm1, Wm2, bm2):
    be = 2000
    return pl.pallas_call(
        _mlp_body,
        grid=(N_EDGES // be,),
        in_specs=[pl.BlockSpec((be, N_GAUSS), lambda i: (i, 0)),
                  pl.BlockSpec((HIDDEN, N_GAUSS), lambda i: (0, 0)),
                  pl.BlockSpec((1, HIDDEN), lambda i: (0, 0)),
                  pl.BlockSpec((HIDDEN, HIDDEN), lambda i: (0, 0)),
                  pl.BlockSpec((1, HIDDEN), lambda i: (0, 0))],
        out_specs=pl.BlockSpec((be, HIDDEN), lambda i: (i, 0)),
        out_shape=jax.ShapeDtypeStruct((N_EDGES, HIDDEN), jnp.float32),
    )(edge_attr, Wm1, bm1.reshape(1, HIDDEN), Wm2, bm2.reshape(1, HIDDEN))


def _sc_body(h_hbm, mlp_hbm, row_hbm, col_hbm, out_hbm,
             aggr_sh, row_v, col_v, hrow_v, hcol_v, mlp_v,
             sem1, sem2, sem3):
    cid = lax.axis_index("c")
    sid = lax.axis_index("s")
    wid = cid * NS + sid

    zeros16 = jnp.zeros((16,), jnp.float32)

    def zbody(j, c):
        for k in range(HIDDEN // 16):
            hrow_v[j, pl.ds(k * 16, 16)] = zeros16
        return c

    lax.fori_loop(0, CHUNK, zbody, 0)

    def zchunk(t, c):
        rc = t * NS + sid

        @pl.when(rc < NRC)
        def _():
            pltpu.sync_copy(hrow_v, aggr_sh.at[pl.ds(rc * CHUNK, CHUNK)])

        return c

    lax.fori_loop(0, (NRC + NS - 1) // NS, zchunk, 0)
    plsc.subcore_barrier()

    ebase0 = wid * EPW

    def chunk_body(ci, c):
        ebase = ebase0 + ci * CHUNK
        pltpu.sync_copy(row_hbm.at[pl.ds(ebase, CHUNK)], row_v)
        pltpu.sync_copy(col_hbm.at[pl.ds(ebase, CHUNK)], col_v)
        cp1 = pltpu.async_copy(h_hbm.at[row_v], hrow_v, sem1)
        cp2 = pltpu.async_copy(h_hbm.at[col_v], hcol_v, sem2)
        cp3 = pltpu.async_copy(mlp_hbm.at[pl.ds(ebase, CHUNK)], mlp_v, sem3)
        cp1.wait()
        cp2.wait()
        cp3.wait()

        def mbody(j, cc):
            for k in range(HIDDEN // 16):
                sl = pl.ds(k * 16, 16)
                m = mlp_v[j, sl]
                hrow_v[j, sl] = hrow_v[j, sl] * m
                hcol_v[j, sl] = hcol_v[j, sl] * m
            return cc

        lax.fori_loop(0, CHUNK, mbody, 0)
        pltpu.sync_copy(hrow_v, aggr_sh.at[col_v], add=True)
        pltpu.sync_copy(hcol_v, aggr_sh.at[row_v], add=True)
        return c

    lax.fori_loop(0, NCHUNK, chunk_body, 0)
    plsc.subcore_barrier()

    def rchunk(t, c):
        rc = t * NS + sid

        @pl.when(rc < NRC)
        def _():
            pltpu.sync_copy(aggr_sh.at[pl.ds(rc * CHUNK, CHUNK)],
                            out_hbm.at[cid, pl.ds(rc * CHUNK, CHUNK)])

        return c

    lax.fori_loop(0, (NRC + NS - 1) // NS, rchunk, 0)


def _sc_aggregate(h, mlp_out, row, col):
    f = pl.kernel(
        _sc_body,
        out_type=jax.ShapeDtypeStruct((NC, N_NODES, HIDDEN), jnp.float32),
        mesh=plsc.VectorSubcoreMesh(core_axis_name="c", subcore_axis_name="s"),
        scratch_types=[
            pltpu.VMEM_SHARED((N_NODES, HIDDEN), jnp.float32),
            pltpu.VMEM((CHUNK,), jnp.int32),
            pltpu.VMEM((CHUNK,), jnp.int32),
            pltpu.VMEM((CHUNK, HIDDEN), jnp.float32),
            pltpu.VMEM((CHUNK, HIDDEN), jnp.float32),
            pltpu.VMEM((CHUNK, HIDDEN), jnp.float32),
            pltpu.SemaphoreType.DMA,
            pltpu.SemaphoreType.DMA,
            pltpu.SemaphoreType.DMA,
        ],
    )
    return f(h, mlp_out, row, col)


def _head_body(p0_ref, p1_ref, w2_ref, b2_ref, wl_ref, bl_ref, o_ref):
    aggr = p0_ref[...] + p1_ref[...]
    t = lax.dot_general(aggr, w2_ref[...], (((1,), (1,)), ((), ())),
                        preferred_element_type=jnp.float32)
    t = _ssp(t + b2_ref[...])
    o = lax.dot_general(t, wl_ref[...], (((1,), (1,)), ((), ())),
                        preferred_element_type=jnp.float32)
    o_ref[...] = o + bl_ref[...]


def _compute_head(p0, p1, W2, b2, Wl, bl):
    bn = 2000
    return pl.pallas_call(
        _head_body,
        grid=(N_NODES // bn,),
        in_specs=[pl.BlockSpec((bn, HIDDEN), lambda i: (i, 0)),
                  pl.BlockSpec((bn, HIDDEN), lambda i: (i, 0)),
                  pl.BlockSpec((HIDDEN, HIDDEN), lambda i: (0, 0)),
                  pl.BlockSpec((1, HIDDEN), lambda i: (0, 0)),
                  pl.BlockSpec((HIDDEN, HIDDEN), lambda i: (0, 0)),
                  pl.BlockSpec((1, HIDDEN), lambda i: (0, 0))],
        out_specs=pl.BlockSpec((bn, HIDDEN), lambda i: (i, 0)),
        out_shape=jax.ShapeDtypeStruct((N_NODES, HIDDEN), jnp.float32),
    )(p0, p1, W2, b2.reshape(1, HIDDEN), Wl, bl.reshape(1, HIDDEN))


def kernel(x, edge_index, edge_weight, edge_attr, W1, W2, b2, Wm1, bm1, Wm2, bm2, Wl, bl):
    row = edge_index[0].astype(jnp.int32)
    col = edge_index[1].astype(jnp.int32)
    h = _compute_h(x, W1)
    mlp_out = _compute_mlp(edge_attr, Wm1, bm1, Wm2, bm2)
    partials = _sc_aggregate(h, mlp_out, row, col)
    return _compute_head(partials[0], partials[1], W2, b2, Wl, bl)


# speedup vs baseline: 4.0826x; 1.1539x over previous
import jax
import jax.numpy as jnp
import numpy as np
from jax import lax
from jax.experimental import pallas as pl
from jax.experimental.pallas import tpu as pltpu
from jax.experimental.pallas import tpu_sc as plsc

N_NODES = 10000
N_EDGES = 320000
HIDDEN = 128
N_GAUSS = 16
SHIFT = float(np.log(2.0))

NC = 2
NS = 16
NW = NC * NS
EPW = N_EDGES // NW
CHUNK = 40
NCHUNK = EPW // CHUNK
NRC = N_NODES // CHUNK

_SETKEYS = ("row", "col", "hrow", "hcol", "mlp", "gsem", "ssem")
NKEY = len(_SETKEYS)


def _ssp(v):
    return jnp.maximum(v, 0.0) + jnp.log1p(jnp.exp(-jnp.abs(v))) - SHIFT


def _h_body(x_ref, w1_ref, o_ref):
    o_ref[...] = lax.dot_general(x_ref[...], w1_ref[...],
                                 (((1,), (1,)), ((), ())),
                                 preferred_element_type=jnp.float32)


def _compute_h(x, W1):
    bn = 2000
    return pl.pallas_call(
        _h_body,
        grid=(N_NODES // bn,),
        in_specs=[pl.BlockSpec((bn, HIDDEN), lambda i: (i, 0)),
                  pl.BlockSpec((HIDDEN, HIDDEN), lambda i: (0, 0))],
        out_specs=pl.BlockSpec((bn, HIDDEN), lambda i: (i, 0)),
        out_shape=jax.ShapeDtypeStruct((N_NODES, HIDDEN), jnp.float32),
    )(x, W1)


def _mlp_body(ea_ref, wm1_ref, bm1_ref, wm2_ref, bm2_ref, o_ref):
    a = lax.dot_general(ea_ref[...], wm1_ref[...], (((1,), (1,)), ((), ())),
                        preferred_element_type=jnp.float32)
    a = _ssp(a + bm1_ref[...])
    o = lax.dot_general(a, wm2_ref[...], (((1,), (1,)), ((), ())),
                        preferred_element_type=jnp.float32)
    o_ref[...] = o + bm2_ref[...]


def _compute_mlp(edge_attr, Wm1, bm1, Wm2, bm2):
    be = 2000
    return pl.pallas_call(
        _mlp_body,
        grid=(N_EDGES // be,),
        in_specs=[pl.BlockSpec((be, N_GAUSS), lambda i: (i, 0)),
                  pl.BlockSpec((HIDDEN, N_GAUSS), lambda i: (0, 0)),
                  pl.BlockSpec((1, HIDDEN), lambda i: (0, 0)),
                  pl.BlockSpec((HIDDEN, HIDDEN), lambda i: (0, 0)),
                  pl.BlockSpec((1, HIDDEN), lambda i: (0, 0))],
        out_specs=pl.BlockSpec((be, HIDDEN), lambda i: (i, 0)),
        out_shape=jax.ShapeDtypeStruct((N_EDGES, HIDDEN), jnp.float32),
    )(edge_attr, Wm1, bm1.reshape(1, HIDDEN), Wm2, bm2.reshape(1, HIDDEN))


def _sc_body(h_hbm, m_hbm, row_hbm, col_hbm, out_hbm, aggr_sh, *bufs):
    A = dict(zip(_SETKEYS, bufs[:NKEY]))
    B = dict(zip(_SETKEYS, bufs[NKEY:]))
    cid = lax.axis_index("c")
    sid = lax.axis_index("s")
    wid = cid * NS + sid
    ebase0 = wid * EPW
    zeros16 = jnp.zeros((16,), jnp.float32)

    def zbody(j, c):
        for k in range(HIDDEN // 16):
            A["hrow"][j, pl.ds(k * 16, 16)] = zeros16
        return c

    lax.fori_loop(0, CHUNK, zbody, 0)

    def zchunk(t, c):
        rc = t * NS + sid

        @pl.when(rc < NRC)
        def _():
            pltpu.sync_copy(A["hrow"], aggr_sh.at[pl.ds(rc * CHUNK, CHUNK)])

        return c

    lax.fori_loop(0, (NRC + NS - 1) // NS, zchunk, 0)
    plsc.subcore_barrier()

    def fire(s, ci):
        ebase = ebase0 + ci * CHUNK
        pltpu.sync_copy(row_hbm.at[pl.ds(ebase, CHUNK)], s["row"])
        pltpu.sync_copy(col_hbm.at[pl.ds(ebase, CHUNK)], s["col"])
        pltpu.async_copy(h_hbm.at[s["row"]], s["hrow"], s["gsem"])
        pltpu.async_copy(h_hbm.at[s["col"]], s["hcol"], s["gsem"])
        pltpu.async_copy(m_hbm.at[pl.ds(ebase, CHUNK)], s["mlp"], s["gsem"])

    def wait_gathers(s, ci):
        ebase = ebase0 + ci * CHUNK
        pltpu.make_async_copy(h_hbm.at[s["row"]], s["hrow"], s["gsem"]).wait()
        pltpu.make_async_copy(h_hbm.at[s["col"]], s["hcol"], s["gsem"]).wait()
        pltpu.make_async_copy(m_hbm.at[pl.ds(ebase, CHUNK)], s["mlp"],
                              s["gsem"]).wait()

    def wait_scatters(s):
        pltpu.make_async_copy(s["hrow"], aggr_sh.at[s["col"]],
                              s["ssem"]).wait()
        pltpu.make_async_copy(s["hcol"], aggr_sh.at[s["row"]],
                              s["ssem"]).wait()

    def compute_and_scatter(s):
        def mbody(j, cc):
            for k in range(HIDDEN // 16):
                sl = pl.ds(k * 16, 16)
                m = s["mlp"][j, sl]
                s["hrow"][j, sl] = s["hrow"][j, sl] * m
                s["hcol"][j, sl] = s["hcol"][j, sl] * m
            return cc

        lax.fori_loop(0, CHUNK, mbody, 0)
        pltpu.async_copy(s["hrow"], aggr_sh.at[s["col"]], s["ssem"],
                         add=True)
        pltpu.async_copy(s["hcol"], aggr_sh.at[s["row"]], s["ssem"],
                         add=True)

    fire(A, 0)
    fire(B, 1)

    def pair_body(t, c):
        wait_gathers(A, 2 * t)
        compute_and_scatter(A)
        wait_gathers(B, 2 * t + 1)
        compute_and_scatter(B)

        @pl.when(2 * t + 2 < NCHUNK)
        def _():
            wait_scatters(A)
            fire(A, 2 * t + 2)

        @pl.when(2 * t + 3 < NCHUNK)
        def _():
            wait_scatters(B)
            fire(B, 2 * t + 3)

        return c

    lax.fori_loop(0, NCHUNK // 2, pair_body, 0)
    wait_scatters(A)
    wait_scatters(B)
    plsc.subcore_barrier()

    def rchunk(t, c):
        rc = t * NS + sid

        @pl.when(rc < NRC)
        def _():
            pltpu.sync_copy(aggr_sh.at[pl.ds(rc * CHUNK, CHUNK)],
                            out_hbm.at[cid, pl.ds(rc * CHUNK, CHUNK)])

        return c

    lax.fori_loop(0, (NRC + NS - 1) // NS, rchunk, 0)


def _sc_aggregate(h, m, row, col):
    f = pl.kernel(
        _sc_body,
        out_type=jax.ShapeDtypeStruct((NC, N_NODES, HIDDEN), jnp.float32),
        mesh=plsc.VectorSubcoreMesh(core_axis_name="c", subcore_axis_name="s"),
        scratch_types=[pltpu.VMEM_SHARED((N_NODES, HIDDEN), jnp.float32)] + 2 * [
            pltpu.VMEM((CHUNK,), jnp.int32),
            pltpu.VMEM((CHUNK,), jnp.int32),
            pltpu.VMEM((CHUNK, HIDDEN), jnp.float32),
            pltpu.VMEM((CHUNK, HIDDEN), jnp.float32),
            pltpu.VMEM((CHUNK, HIDDEN), jnp.float32),
            pltpu.SemaphoreType.DMA,
            pltpu.SemaphoreType.DMA,
        ],
    )
    return f(h, m, row, col)


def _head_body(p0_ref, p1_ref, w2_ref, b2_ref, wl_ref, bl_ref, o_ref):
    aggr = p0_ref[...] + p1_ref[...]
    t = lax.dot_general(aggr, w2_ref[...], (((1,), (1,)), ((), ())),
                        preferred_element_type=jnp.float32)
    t = _ssp(t + b2_ref[...])
    o = lax.dot_general(t, wl_ref[...], (((1,), (1,)), ((), ())),
                        preferred_element_type=jnp.float32)
    o_ref[...] = o + bl_ref[...]


def _compute_head(p, W2, b2, Wl, bl):
    bn = 2000
    full_spec = pl.BlockSpec((bn, HIDDEN), lambda i: (i, 0))
    return pl.pallas_call(
        _head_body,
        grid=(N_NODES // bn,),
        in_specs=[full_spec, full_spec,
                  pl.BlockSpec((HIDDEN, HIDDEN), lambda i: (0, 0)),
                  pl.BlockSpec((1, HIDDEN), lambda i: (0, 0)),
                  pl.BlockSpec((HIDDEN, HIDDEN), lambda i: (0, 0)),
                  pl.BlockSpec((1, HIDDEN), lambda i: (0, 0))],
        out_specs=pl.BlockSpec((bn, HIDDEN), lambda i: (i, 0)),
        out_shape=jax.ShapeDtypeStruct((N_NODES, HIDDEN), jnp.float32),
    )(p[0], p[1], W2, b2.reshape(1, HIDDEN), Wl, bl.reshape(1, HIDDEN))


def kernel(x, edge_index, edge_weight, edge_attr, W1, W2, b2, Wm1, bm1, Wm2, bm2, Wl, bl):
    row = edge_index[0].astype(jnp.int32)
    col = edge_index[1].astype(jnp.int32)
    h = _compute_h(x, W1)
    m = _compute_mlp(edge_attr, Wm1, bm1, Wm2, bm2)
    partials = _sc_aggregate(h, m, row, col)
    return _compute_head(partials, W2, b2, Wl, bl)


# speedup vs baseline: 4.7036x; 1.1521x over previous
import jax
import jax.numpy as jnp
import numpy as np
from jax import lax
from jax.experimental import pallas as pl
from jax.experimental.pallas import tpu as pltpu
from jax.experimental.pallas import tpu_sc as plsc

N_NODES = 10000
N_EDGES = 320000
HIDDEN = 128
N_GAUSS = 16
SHIFT = float(np.log(2.0))

NC = 2
NS = 16
NW = NC * NS
EPW = N_EDGES // NW
CHUNK = 40
NCHUNK = EPW // CHUNK
NRC = N_NODES // CHUNK

IDXBLK = 10
BLKE = IDXBLK * CHUNK
NBLK = NCHUNK // IDXBLK

_SETKEYS = ("hrow", "hcol", "mlp", "gsem", "ssem")
NKEY = len(_SETKEYS)


def _ssp(v):
    return jnp.maximum(v, 0.0) + jnp.log1p(jnp.exp(-jnp.abs(v))) - SHIFT


def _h_body(x_ref, w1_ref, o_ref):
    o_ref[...] = lax.dot_general(x_ref[...], w1_ref[...],
                                 (((1,), (1,)), ((), ())),
                                 preferred_element_type=jnp.float32)


def _compute_h(x, W1):
    bn = 2000
    return pl.pallas_call(
        _h_body,
        grid=(N_NODES // bn,),
        in_specs=[pl.BlockSpec((bn, HIDDEN), lambda i: (i, 0)),
                  pl.BlockSpec((HIDDEN, HIDDEN), lambda i: (0, 0))],
        out_specs=pl.BlockSpec((bn, HIDDEN), lambda i: (i, 0)),
        out_shape=jax.ShapeDtypeStruct((N_NODES, HIDDEN), jnp.float32),
    )(x, W1)


def _mlp_body(ea_ref, wm1_ref, bm1_ref, wm2_ref, bm2_ref, o_ref):
    a = lax.dot_general(ea_ref[...], wm1_ref[...], (((1,), (1,)), ((), ())),
                        preferred_element_type=jnp.float32)
    a = _ssp(a + bm1_ref[...])
    o = lax.dot_general(a, wm2_ref[...], (((1,), (1,)), ((), ())),
                        preferred_element_type=jnp.float32)
    o_ref[...] = o + bm2_ref[...]


def _compute_mlp(edge_attr, Wm1, bm1, Wm2, bm2):
    be = 2000
    return pl.pallas_call(
        _mlp_body,
        grid=(N_EDGES // be,),
        in_specs=[pl.BlockSpec((be, N_GAUSS), lambda i: (i, 0)),
                  pl.BlockSpec((HIDDEN, N_GAUSS), lambda i: (0, 0)),
                  pl.BlockSpec((1, HIDDEN), lambda i: (0, 0)),
                  pl.BlockSpec((HIDDEN, HIDDEN), lambda i: (0, 0)),
                  pl.BlockSpec((1, HIDDEN), lambda i: (0, 0))],
        out_specs=pl.BlockSpec((be, HIDDEN), lambda i: (i, 0)),
        out_shape=jax.ShapeDtypeStruct((N_EDGES, HIDDEN), jnp.float32),
    )(edge_attr, Wm1, bm1.reshape(1, HIDDEN), Wm2, bm2.reshape(1, HIDDEN))


def _sc_body(h_hbm, m_hbm, row_hbm, col_hbm, out_hbm, aggr_sh, *bufs):
    A = dict(zip(_SETKEYS, bufs[:NKEY]))
    B = dict(zip(_SETKEYS, bufs[NKEY:2 * NKEY]))
    rowblk0, colblk0, rowblk1, colblk1, isem = bufs[2 * NKEY:]
    rowblks = (rowblk0, rowblk1)
    colblks = (colblk0, colblk1)
    cid = lax.axis_index("c")
    sid = lax.axis_index("s")
    wid = cid * NS + sid
    ebase0 = wid * EPW
    zeros16 = jnp.zeros((16,), jnp.float32)

    def zbody(j, c):
        for k in range(HIDDEN // 16):
            A["hrow"][j, pl.ds(k * 16, 16)] = zeros16
        return c

    lax.fori_loop(0, CHUNK, zbody, 0)

    def zchunk(t, c):
        rc = t * NS + sid

        @pl.when(rc < NRC)
        def _():
            pltpu.sync_copy(A["hrow"], aggr_sh.at[pl.ds(rc * CHUNK, CHUNK)])

        return c

    lax.fori_loop(0, (NRC + NS - 1) // NS, zchunk, 0)
    plsc.subcore_barrier()

    def fire_idx(b):
        b = jnp.int32(b)
        ebase = ebase0 + b * BLKE

        @pl.when(lax.rem(b, 2) == 0)
        def _():
            pltpu.async_copy(row_hbm.at[pl.ds(ebase, BLKE)], rowblk0, isem)
            pltpu.async_copy(col_hbm.at[pl.ds(ebase, BLKE)], colblk0, isem)

        @pl.when(lax.rem(b, 2) == 1)
        def _():
            pltpu.async_copy(row_hbm.at[pl.ds(ebase, BLKE)], rowblk1, isem)
            pltpu.async_copy(col_hbm.at[pl.ds(ebase, BLKE)], colblk1, isem)

    def wait_idx():
        pltpu.make_async_copy(row_hbm.at[pl.ds(0, BLKE)], rowblk0,
                              isem).wait()
        pltpu.make_async_copy(col_hbm.at[pl.ds(0, BLKE)], colblk0,
                              isem).wait()

    def with_blk(ci, fn):
        ci = jnp.int32(ci)
        off = lax.rem(ci, IDXBLK) * CHUNK
        par = lax.rem(ci // IDXBLK, 2)

        @pl.when(par == 0)
        def _():
            fn(rowblk0.at[pl.ds(off, CHUNK)], colblk0.at[pl.ds(off, CHUNK)])

        @pl.when(par == 1)
        def _():
            fn(rowblk1.at[pl.ds(off, CHUNK)], colblk1.at[pl.ds(off, CHUNK)])

    def fire(s, ci):
        def go(rows, cols):
            pltpu.async_copy(h_hbm.at[rows], s["hrow"], s["gsem"])
            pltpu.async_copy(h_hbm.at[cols], s["hcol"], s["gsem"])
            pltpu.async_copy(m_hbm.at[pl.ds(ebase0 + ci * CHUNK, CHUNK)],
                             s["mlp"], s["gsem"])

        with_blk(ci, go)

    def wait_gathers(s, ci):
        def go(rows, cols):
            pltpu.make_async_copy(h_hbm.at[rows], s["hrow"], s["gsem"]).wait()
            pltpu.make_async_copy(h_hbm.at[cols], s["hcol"], s["gsem"]).wait()
            pltpu.make_async_copy(m_hbm.at[pl.ds(ebase0 + ci * CHUNK, CHUNK)],
                                  s["mlp"], s["gsem"]).wait()

        with_blk(ci, go)

    def wait_scatters(s, ci):
        def go(rows, cols):
            pltpu.make_async_copy(s["hrow"], aggr_sh.at[cols],
                                  s["ssem"]).wait()
            pltpu.make_async_copy(s["hcol"], aggr_sh.at[rows],
                                  s["ssem"]).wait()

        with_blk(ci, go)

    def compute_and_scatter(s, ci):
        def mbody(j, cc):
            for jj in range(2):
                for k in range(HIDDEN // 16):
                    sl = pl.ds(k * 16, 16)
                    m = s["mlp"][2 * j + jj, sl]
                    s["hrow"][2 * j + jj, sl] = s["hrow"][2 * j + jj, sl] * m
                    s["hcol"][2 * j + jj, sl] = s["hcol"][2 * j + jj, sl] * m
            return cc

        lax.fori_loop(0, CHUNK // 2, mbody, 0)

        def go(rows, cols):
            pltpu.async_copy(s["hrow"], aggr_sh.at[cols], s["ssem"], add=True)
            pltpu.async_copy(s["hcol"], aggr_sh.at[rows], s["ssem"], add=True)

        with_blk(ci, go)

    fire_idx(0)
    fire_idx(1)
    wait_idx()
    wait_idx()
    fire(A, 0)
    fire(B, 1)

    def pair_body(t, c):
        ci = 2 * t
        dec = lax.rem(ci, IDXBLK)

        @pl.when(jnp.logical_and(dec == 0, ci // IDXBLK + 1 < NBLK))
        def _():
            fire_idx(ci // IDXBLK + 1)

        wait_gathers(A, ci)
        compute_and_scatter(A, ci)
        wait_gathers(B, ci + 1)
        compute_and_scatter(B, ci + 1)

        @pl.when(jnp.logical_and(dec == IDXBLK - 2,
                                 ci // IDXBLK + 1 < NBLK))
        def _():
            wait_idx()

        @pl.when(ci + 2 < NCHUNK)
        def _():
            wait_scatters(A, ci)
            fire(A, ci + 2)

        @pl.when(ci + 3 < NCHUNK)
        def _():
            wait_scatters(B, ci + 1)
            fire(B, ci + 3)

        return c

    lax.fori_loop(0, NCHUNK // 2, pair_body, 0)
    wait_scatters(A, NCHUNK - 2)
    wait_scatters(B, NCHUNK - 1)
    plsc.subcore_barrier()

    def rchunk(t, c):
        rc = t * NS + sid

        @pl.when(rc < NRC)
        def _():
            pltpu.sync_copy(aggr_sh.at[pl.ds(rc * CHUNK, CHUNK)],
                            out_hbm.at[cid, pl.ds(rc * CHUNK, CHUNK)])

        return c

    lax.fori_loop(0, (NRC + NS - 1) // NS, rchunk, 0)


def _sc_aggregate(h, m, row, col):
    f = pl.kernel(
        _sc_body,
        out_type=jax.ShapeDtypeStruct((NC, N_NODES, HIDDEN), jnp.float32),
        mesh=plsc.VectorSubcoreMesh(core_axis_name="c", subcore_axis_name="s"),
        scratch_types=[pltpu.VMEM_SHARED((N_NODES, HIDDEN), jnp.float32)] + 2 * [
            pltpu.VMEM((CHUNK, HIDDEN), jnp.float32),
            pltpu.VMEM((CHUNK, HIDDEN), jnp.float32),
            pltpu.VMEM((CHUNK, HIDDEN), jnp.float32),
            pltpu.SemaphoreType.DMA,
            pltpu.SemaphoreType.DMA,
        ] + [
            pltpu.VMEM((BLKE,), jnp.int32),
            pltpu.VMEM((BLKE,), jnp.int32),
            pltpu.VMEM((BLKE,), jnp.int32),
            pltpu.VMEM((BLKE,), jnp.int32),
            pltpu.SemaphoreType.DMA,
        ],
    )
    return f(h, m, row, col)


def _head_body(p0_ref, p1_ref, w2_ref, b2_ref, wl_ref, bl_ref, o_ref):
    aggr = p0_ref[...] + p1_ref[...]
    t = lax.dot_general(aggr, w2_ref[...], (((1,), (1,)), ((), ())),
                        preferred_element_type=jnp.float32)
    t = _ssp(t + b2_ref[...])
    o = lax.dot_general(t, wl_ref[...], (((1,), (1,)), ((), ())),
                        preferred_element_type=jnp.float32)
    o_ref[...] = o + bl_ref[...]


def _compute_head(p, W2, b2, Wl, bl):
    bn = 2000
    full_spec = pl.BlockSpec((bn, HIDDEN), lambda i: (i, 0))
    return pl.pallas_call(
        _head_body,
        grid=(N_NODES // bn,),
        in_specs=[full_spec, full_spec,
                  pl.BlockSpec((HIDDEN, HIDDEN), lambda i: (0, 0)),
                  pl.BlockSpec((1, HIDDEN), lambda i: (0, 0)),
                  pl.BlockSpec((HIDDEN, HIDDEN), lambda i: (0, 0)),
                  pl.BlockSpec((1, HIDDEN), lambda i: (0, 0))],
        out_specs=pl.BlockSpec((bn, HIDDEN), lambda i: (i, 0)),
        out_shape=jax.ShapeDtypeStruct((N_NODES, HIDDEN), jnp.float32),
    )(p[0], p[1], W2, b2.reshape(1, HIDDEN), Wl, bl.reshape(1, HIDDEN))


def kernel(x, edge_index, edge_weight, edge_attr, W1, W2, b2, Wm1, bm1, Wm2, bm2, Wl, bl):
    row = edge_index[0].astype(jnp.int32)
    col = edge_index[1].astype(jnp.int32)
    h = _compute_h(x, W1)
    m = _compute_mlp(edge_attr, Wm1, bm1, Wm2, bm2)
    partials = _sc_aggregate(h, m, row, col)
    return _compute_head(partials, W2, b2, Wl, bl)
